# Initial kernel scaffold; baseline (speedup 1.0000x reference)
import jax
import jax.numpy as jnp
from jax.experimental import pallas as pl


def kernel(P_batch, Q_batch):
    raise NotImplementedError("write your pallas kernel here")


# speedup vs baseline: 4.6666x; 4.6666x over previous
import jax
import jax.numpy as jnp
from jax.experimental import pallas as pl
from jax.experimental.pallas import tpu as pltpu

_NPROJ = 50
_N = 256
_DIM = 3
_BATCH = 8
_LOG2N = 8


def _projections():
    z = jax.random.normal(jax.random.key(42), (_NPROJ, _DIM, 2), dtype=jnp.float32)
    q, _ = jnp.linalg.qr(z)
    return q


def _bitonic_sort_rows(x):
    n = x.shape[-1]
    lane = jax.lax.broadcasted_iota(jnp.int32, x.shape, x.ndim - 1)
    k = 2
    while k <= n:
        j = k // 2
        while j >= 1:
            upper = (lane & j) != 0
            partner = jnp.where(upper, jnp.roll(x, j, axis=-1),
                                jnp.roll(x, -j, axis=-1))
            asc = (lane & k) == 0
            take_min = asc != upper
            x = jnp.where(take_min, jnp.minimum(x, partner),
                          jnp.maximum(x, partner))
            j //= 2
        k *= 2
    return x


def _ssw_kernel(pt_ref, qt_ref, pa_ref, pb_ref, out_ref):
    pt = pt_ref[0]
    qt = qt_ref[0]
    pa = pa_ref[...]
    pb = pb_ref[...]

    def coords(xt):
        xa = (pa[:, 0:1] * xt[0:1, :] + pa[:, 1:2] * xt[1:2, :]
              + pa[:, 2:3] * xt[2:3, :])
        xb = (pb[:, 0:1] * xt[0:1, :] + pb[:, 1:2] * xt[1:2, :]
              + pb[:, 2:3] * xt[2:3, :])
        return (jnp.pi + jnp.arctan2(-xb, -xa)) * (1.0 / (2.0 * jnp.pi))

    u = _bitonic_sort_rows(coords(pt))
    v = _bitonic_sort_rows(coords(qt))

    total = jnp.zeros((1, 1), jnp.float32)
    for l in range(_NPROJ):
        circ = v[l:l + 1, :]
        for m in range(_LOG2N):
            circ = jnp.concatenate(
                [circ, jnp.roll(circ, -(1 << m), axis=1)], axis=0)
        diff = jnp.abs(u[l:l + 1, :] - circ)
        d = jnp.minimum(diff, 1.0 - diff)
        cost = jnp.sum(d * d, axis=1, keepdims=True)
        total = total + jnp.min(cost, axis=(0, 1), keepdims=True)
    out_ref[0] = jnp.sqrt(total * (1.0 / (_NPROJ * _N)))


def kernel(P_batch, Q_batch):
    proj = _projections()
    pa = proj[:, :, 0]
    pb = proj[:, :, 1]
    pt = P_batch.transpose(0, 2, 1)
    qt = Q_batch.transpose(0, 2, 1)

    losses = pl.pallas_call(
        _ssw_kernel,
        grid=(_BATCH,),
        in_specs=[
            pl.BlockSpec((1, _DIM, _N), lambda b: (b, 0, 0)),
            pl.BlockSpec((1, _DIM, _N), lambda b: (b, 0, 0)),
            pl.BlockSpec((_NPROJ, _DIM), lambda b: (0, 0)),
            pl.BlockSpec((_NPROJ, _DIM), lambda b: (0, 0)),
        ],
        out_specs=pl.BlockSpec((1, 1, 1), lambda b: (b, 0, 0)),
        out_shape=jax.ShapeDtypeStruct((_BATCH, 1, 1), jnp.float32),
        compiler_params=pltpu.CompilerParams(
            dimension_semantics=("arbitrary",)),
    )(pt, qt, pa, pb)
    return jnp.sum(losses)


# speedup vs baseline: 9.4824x; 2.0320x over previous
import jax
import jax.numpy as jnp
from jax.experimental import pallas as pl
from jax.experimental.pallas import tpu as pltpu

_NPROJ = 50
_N = 256
_DIM = 3
_BATCH = 8
_LOG2N = 8


def _projections():
    z = jax.random.normal(jax.random.key(42), (_NPROJ, _DIM, 2), dtype=jnp.float32)
    q, _ = jnp.linalg.qr(z)
    return q


def _bitonic_sort_rows(x):
    n = x.shape[-1]
    lane = jax.lax.broadcasted_iota(jnp.int32, x.shape, x.ndim - 1)
    k = 2
    while k <= n:
        j = k // 2
        while j >= 1:
            upper = (lane & j) != 0
            partner = jnp.where(upper, jnp.roll(x, j, axis=-1),
                                jnp.roll(x, -j, axis=-1))
            asc = (lane & k) == 0
            take_min = asc != upper
            x = jnp.where(take_min, jnp.minimum(x, partner),
                          jnp.maximum(x, partner))
            j //= 2
        k *= 2
    return x


def _ssw_kernel(pt_ref, qt_ref, pa_ref, pb_ref, out_ref):
    pt = pt_ref[0]
    qt = qt_ref[0]
    pa = pa_ref[...]
    pb = pb_ref[...]

    def coords(xt):
        xa = (pa[:, 0:1] * xt[0:1, :] + pa[:, 1:2] * xt[1:2, :]
              + pa[:, 2:3] * xt[2:3, :])
        xb = (pb[:, 0:1] * xt[0:1, :] + pb[:, 1:2] * xt[1:2, :]
              + pb[:, 2:3] * xt[2:3, :])
        return (jnp.pi + jnp.arctan2(-xb, -xa)) * (1.0 / (2.0 * jnp.pi))

    u = _bitonic_sort_rows(coords(pt))
    v = _bitonic_sort_rows(coords(qt))

    total = jnp.zeros((1, 1), jnp.float32)
    for l in range(_NPROJ):
        vb = jnp.broadcast_to(v[l:l + 1, :], (_N, _N))
        circ = pltpu.roll(vb, 0, axis=1, stride=1, stride_axis=0)
        diff = jnp.abs(u[l:l + 1, :] - circ)
        d = jnp.minimum(diff, 1.0 - diff)
        cost = jnp.sum(d * d, axis=1, keepdims=True)
        total = total + jnp.min(cost, axis=(0, 1), keepdims=True)
    out_ref[0] = jnp.sqrt(total * (1.0 / (_NPROJ * _N)))


def kernel(P_batch, Q_batch):
    proj = _projections()
    pa = proj[:, :, 0]
    pb = proj[:, :, 1]
    pt = P_batch.transpose(0, 2, 1)
    qt = Q_batch.transpose(0, 2, 1)

    losses = pl.pallas_call(
        _ssw_kernel,
        grid=(_BATCH,),
        in_specs=[
            pl.BlockSpec((1, _DIM, _N), lambda b: (b, 0, 0)),
            pl.BlockSpec((1, _DIM, _N), lambda b: (b, 0, 0)),
            pl.BlockSpec((_NPROJ, _DIM), lambda b: (0, 0)),
            pl.BlockSpec((_NPROJ, _DIM), lambda b: (0, 0)),
        ],
        out_specs=pl.BlockSpec((1, 1, 1), lambda b: (b, 0, 0)),
        out_shape=jax.ShapeDtypeStruct((_BATCH, 1, 1), jnp.float32),
        compiler_params=pltpu.CompilerParams(
            dimension_semantics=("parallel",)),
    )(pt, qt, pa, pb)
    return jnp.sum(losses)


# speedup vs baseline: 13.1528x; 1.3871x over previous
import jax
import jax.numpy as jnp
from jax.experimental import pallas as pl
from jax.experimental.pallas import tpu as pltpu

_NPROJ = 50
_N = 256
_DIM = 3
_BATCH = 8
_LOG2N = 8


def _projections():
    z = jax.random.normal(jax.random.key(42), (_NPROJ, _DIM, 2), dtype=jnp.float32)
    q, _ = jnp.linalg.qr(z)
    return q


def _bitonic_sort_rows(x):
    n = x.shape[-1]
    lane = jax.lax.broadcasted_iota(jnp.int32, x.shape, x.ndim - 1)
    k = 2
    while k <= n:
        j = k // 2
        while j >= 1:
            upper = (lane & j) != 0
            partner = jnp.where(upper, jnp.roll(x, j, axis=-1),
                                jnp.roll(x, -j, axis=-1))
            asc = (lane & k) == 0
            take_min = asc != upper
            x = jnp.where(take_min, jnp.minimum(x, partner),
                          jnp.maximum(x, partner))
            j //= 2
        k *= 2
    return x


def _ssw_kernel(pt_ref, qt_ref, pa_ref, pb_ref, out_ref):
    pt = pt_ref[0]
    qt = qt_ref[0]
    pa = pa_ref[...]
    pb = pb_ref[...]

    def coords(xt):
        xa = (pa[:, 0:1] * xt[0:1, :] + pa[:, 1:2] * xt[1:2, :]
              + pa[:, 2:3] * xt[2:3, :])
        xb = (pb[:, 0:1] * xt[0:1, :] + pb[:, 1:2] * xt[1:2, :]
              + pb[:, 2:3] * xt[2:3, :])
        return (jnp.pi + jnp.arctan2(-xb, -xa)) * (1.0 / (2.0 * jnp.pi))

    u = _bitonic_sort_rows(coords(pt))
    v = _bitonic_sort_rows(coords(qt))

    ones = jnp.ones((_N, 1), jnp.float32)
    total = jnp.zeros((1, 1), jnp.float32)
    for l in range(_NPROJ):
        vb = jnp.broadcast_to(v[l:l + 1, :], (_N, _N))
        circ = pltpu.roll(vb, 0, axis=1, stride=1, stride_axis=0)
        diff = jnp.abs(u[l:l + 1, :] - circ)
        d = jnp.minimum(diff, 1.0 - diff)
        cost = jax.lax.dot(d * d, ones,
                           preferred_element_type=jnp.float32)
        total = total + jnp.min(cost, axis=(0, 1), keepdims=True)
    out_ref[0] = jnp.sqrt(total * (1.0 / (_NPROJ * _N)))


def kernel(P_batch, Q_batch):
    proj = _projections()
    pa = proj[:, :, 0]
    pb = proj[:, :, 1]
    pt = P_batch.transpose(0, 2, 1)
    qt = Q_batch.transpose(0, 2, 1)

    losses = pl.pallas_call(
        _ssw_kernel,
        grid=(_BATCH,),
        in_specs=[
            pl.BlockSpec((1, _DIM, _N), lambda b: (b, 0, 0)),
            pl.BlockSpec((1, _DIM, _N), lambda b: (b, 0, 0)),
            pl.BlockSpec((_NPROJ, _DIM), lambda b: (0, 0)),
            pl.BlockSpec((_NPROJ, _DIM), lambda b: (0, 0)),
        ],
        out_specs=pl.BlockSpec((1, 1, 1), lambda b: (b, 0, 0)),
        out_shape=jax.ShapeDtypeStruct((_BATCH, 1, 1), jnp.float32),
        compiler_params=pltpu.CompilerParams(
            dimension_semantics=("parallel",)),
    )(pt, qt, pa, pb)
    return jnp.sum(losses)


# speedup vs baseline: 14.8782x; 1.1312x over previous
import numpy as np

import jax
import jax.numpy as jnp
from jax.experimental import pallas as pl
from jax.experimental.pallas import tpu as pltpu

_NPROJ = 50
_N = 256
_DIM = 3
_BATCH = 8
_LOG2N = 8


def _projections():
    z = jax.random.normal(jax.random.key(42), (_NPROJ, _DIM, 2), dtype=jnp.float32)
    q, _ = jnp.linalg.qr(z)
    return np.asarray(q)


_PROJ_CONST = _projections()


def _bitonic_sort_rows(x):
    n = x.shape[-1]
    lane = jax.lax.broadcasted_iota(jnp.int32, x.shape, x.ndim - 1)
    k = 2
    while k <= n:
        j = k // 2
        while j >= 1:
            upper = (lane & j) != 0
            partner = jnp.where(upper, jnp.roll(x, j, axis=-1),
                                jnp.roll(x, -j, axis=-1))
            asc = (lane & k) == 0
            take_min = asc != upper
            x = jnp.where(take_min, jnp.minimum(x, partner),
                          jnp.maximum(x, partner))
            j //= 2
        k *= 2
    return x


def _ssw_kernel(pt_ref, qt_ref, pa_ref, pb_ref, out_ref):
    pt = pt_ref[0]
    qt = qt_ref[0]
    pa = pa_ref[...]
    pb = pb_ref[...]

    def coords(xt):
        xa = (pa[:, 0:1] * xt[0:1, :] + pa[:, 1:2] * xt[1:2, :]
              + pa[:, 2:3] * xt[2:3, :])
        xb = (pb[:, 0:1] * xt[0:1, :] + pb[:, 1:2] * xt[1:2, :]
              + pb[:, 2:3] * xt[2:3, :])
        return (jnp.pi + jnp.arctan2(-xb, -xa)) * (1.0 / (2.0 * jnp.pi))

    u = _bitonic_sort_rows(coords(pt))
    v = _bitonic_sort_rows(coords(qt))

    ones = jnp.ones((_N, 1), jnp.float32)
    total = jnp.zeros((1, 1), jnp.float32)
    for l in range(_NPROJ):
        vb = jnp.broadcast_to(v[l:l + 1, :], (_N, _N))
        circ = pltpu.roll(vb, 0, axis=1, stride=1, stride_axis=0)
        diff = jnp.abs(u[l:l + 1, :] - circ)
        d = jnp.minimum(diff, 1.0 - diff)
        cost = jax.lax.dot(d * d, ones,
                           preferred_element_type=jnp.float32)
        total = total + jnp.min(cost, axis=(0, 1), keepdims=True)
    out_ref[0] = jnp.sqrt(total * (1.0 / (_NPROJ * _N)))


def kernel(P_batch, Q_batch):
    pa = jnp.asarray(_PROJ_CONST[:, :, 0])
    pb = jnp.asarray(_PROJ_CONST[:, :, 1])
    pt = P_batch.transpose(0, 2, 1)
    qt = Q_batch.transpose(0, 2, 1)

    losses = pl.pallas_call(
        _ssw_kernel,
        grid=(_BATCH,),
        in_specs=[
            pl.BlockSpec((1, _DIM, _N), lambda b: (b, 0, 0)),
            pl.BlockSpec((1, _DIM, _N), lambda b: (b, 0, 0)),
            pl.BlockSpec((_NPROJ, _DIM), lambda b: (0, 0)),
            pl.BlockSpec((_NPROJ, _DIM), lambda b: (0, 0)),
        ],
        out_specs=pl.BlockSpec((1, 1, 1), lambda b: (b, 0, 0)),
        out_shape=jax.ShapeDtypeStruct((_BATCH, 1, 1), jnp.float32),
        compiler_params=pltpu.CompilerParams(
            dimension_semantics=("parallel",)),
    )(pt, qt, pa, pb)
    return jnp.sum(losses)


# speedup vs baseline: 14.9160x; 1.0025x over previous
import numpy as np

import jax
import jax.numpy as jnp
from jax.experimental import pallas as pl
from jax.experimental.pallas import tpu as pltpu

_NPROJ = 50
_N = 256
_DIM = 3
_BATCH = 8
_LOG2N = 8
_ITEMS_PER_STEP = 1


def _projections():
    z = jax.random.normal(jax.random.key(42), (_NPROJ, _DIM, 2), dtype=jnp.float32)
    q, _ = jnp.linalg.qr(z)
    return q


try:
    _PROJ_CONST = np.asarray(_projections())
except Exception:
    _PROJ_CONST = None


def _bitonic_sort_rows(x):
    n = x.shape[-1]
    lane = jax.lax.broadcasted_iota(jnp.int32, x.shape, x.ndim - 1)
    k = 2
    while k <= n:
        j = k // 2
        while j >= 1:
            upper = (lane & j) != 0
            partner = jnp.where(upper, jnp.roll(x, j, axis=-1),
                                jnp.roll(x, -j, axis=-1))
            asc = (lane & k) == 0
            take_min = asc != upper
            x = jnp.where(take_min, jnp.minimum(x, partner),
                          jnp.maximum(x, partner))
            j //= 2
        k *= 2
    return x


def _ssw_kernel(pt_ref, qt_ref, pa_ref, pb_ref, out_ref):
    pa = pa_ref[...]
    pb = pb_ref[...]

    def coords(xt):
        xa = (pa[:, 0:1] * xt[0:1, :] + pa[:, 1:2] * xt[1:2, :]
              + pa[:, 2:3] * xt[2:3, :])
        xb = (pb[:, 0:1] * xt[0:1, :] + pb[:, 1:2] * xt[1:2, :]
              + pb[:, 2:3] * xt[2:3, :])
        return (jnp.pi + jnp.arctan2(-xb, -xa)) * (1.0 / (2.0 * jnp.pi))

    ones = jnp.ones((_N, 1), jnp.float32)

    def shift_costs(ur, vb, shift, stride):
        circ = pltpu.roll(vb, shift, axis=1, stride=stride, stride_axis=0)
        diff = jnp.abs(ur - circ)
        d = jnp.minimum(diff, 1.0 - diff)
        return jax.lax.dot(d * d, ones,
                           preferred_element_type=jnp.float32)

    uv = _bitonic_sort_rows(
        jnp.concatenate([coords(pt_ref[0]), coords(qt_ref[0])], axis=0))
    u = uv[:_NPROJ]
    v = uv[_NPROJ:]

    total = jnp.zeros((1, 1), jnp.float32)
    for l in range(_NPROJ):
        vb = jnp.broadcast_to(v[l:l + 1, :], (_N, _N))
        fr = shift_costs(u[l:l + 1, :], vb, 0, 1)
        total = total + jnp.min(fr, axis=(0, 1), keepdims=True)
    out_ref[0] = jnp.sqrt(total * (1.0 / (_NPROJ * _N)))


def kernel(P_batch, Q_batch):
    proj = _projections() if _PROJ_CONST is None else jnp.asarray(_PROJ_CONST)
    pa = proj[:, :, 0]
    pb = proj[:, :, 1]
    pt = P_batch.transpose(0, 2, 1)
    qt = Q_batch.transpose(0, 2, 1)

    losses = pl.pallas_call(
        _ssw_kernel,
        grid=(_BATCH,),
        in_specs=[
            pl.BlockSpec((1, _DIM, _N), lambda b: (b, 0, 0)),
            pl.BlockSpec((1, _DIM, _N), lambda b: (b, 0, 0)),
            pl.BlockSpec((_NPROJ, _DIM), lambda b: (0, 0)),
            pl.BlockSpec((_NPROJ, _DIM), lambda b: (0, 0)),
        ],
        out_specs=pl.BlockSpec((1, 1, 1), lambda b: (b, 0, 0)),
        out_shape=jax.ShapeDtypeStruct((_BATCH, 1, 1), jnp.float32),
        compiler_params=pltpu.CompilerParams(
            dimension_semantics=("parallel",)),
    )(pt, qt, pa, pb)
    return jnp.sum(losses)


# speedup vs baseline: 14.9399x; 1.0016x over previous
import numpy as np

import jax
import jax.numpy as jnp
from jax.experimental import pallas as pl
from jax.experimental.pallas import tpu as pltpu

_NPROJ = 50
_N = 256
_DIM = 3
_BATCH = 8


def _projections():
    z = jax.random.normal(jax.random.key(42), (_NPROJ, _DIM, 2), dtype=jnp.float32)
    q, _ = jnp.linalg.qr(z)
    return q


try:
    _PROJ_CONST = np.asarray(_projections())
except Exception:
    _PROJ_CONST = None


def _bitonic_sort_rows(x):
    n = x.shape[-1]
    lane = jax.lax.broadcasted_iota(jnp.int32, x.shape, x.ndim - 1)
    k = 2
    while k <= n:
        j = k // 2
        while j >= 1:
            upper = (lane & j) != 0
            partner = jnp.where(upper, jnp.roll(x, j, axis=-1),
                                jnp.roll(x, -j, axis=-1))
            asc = (lane & k) == 0
            take_min = asc != upper
            x = jnp.where(take_min, jnp.minimum(x, partner),
                          jnp.maximum(x, partner))
            j //= 2
        k *= 2
    return x


def _ssw_kernel(pt_ref, qt_ref, pa_ref, pb_ref, out_ref):
    pa = pa_ref[...]
    pb = pb_ref[...]

    def coords(xt):
        xa = (pa[:, 0:1] * xt[0:1, :] + pa[:, 1:2] * xt[1:2, :]
              + pa[:, 2:3] * xt[2:3, :])
        xb = (pb[:, 0:1] * xt[0:1, :] + pb[:, 1:2] * xt[1:2, :]
              + pb[:, 2:3] * xt[2:3, :])
        return (jnp.pi + jnp.arctan2(-xb, -xa)) * (1.0 / (2.0 * jnp.pi))

    ones = jnp.ones((_N, 1), jnp.float32)

    def shift_costs(ur, vb, shift, stride):
        circ = pltpu.roll(vb, shift, axis=1, stride=stride, stride_axis=0)
        diff = jnp.abs(ur - circ)
        d = jnp.minimum(diff, 1.0 - diff)
        return jax.lax.dot(d * d, ones,
                           preferred_element_type=jnp.float32)

    uv = _bitonic_sort_rows(
        jnp.concatenate([coords(pt_ref[0]), coords(qt_ref[0])], axis=0))
    u = uv[:_NPROJ]
    v = uv[_NPROJ:]

    total = jnp.zeros((1, 1), jnp.float32)
    for l in range(_NPROJ):
        vb = jnp.broadcast_to(v[l:l + 1, :], (_N, _N))
        fr = shift_costs(u[l:l + 1, :], vb, 0, 1)
        total = total + jnp.min(fr, axis=(0, 1), keepdims=True)
    out_ref[0] = jnp.sqrt(total * (1.0 / (_NPROJ * _N)))


def kernel(P_batch, Q_batch):
    proj = _projections() if _PROJ_CONST is None else jnp.asarray(_PROJ_CONST)
    pa = proj[:, :, 0]
    pb = proj[:, :, 1]
    pt = P_batch.transpose(0, 2, 1)
    qt = Q_batch.transpose(0, 2, 1)

    losses = pl.pallas_call(
        _ssw_kernel,
        grid=(_BATCH,),
        in_specs=[
            pl.BlockSpec((1, _DIM, _N), lambda b: (b, 0, 0)),
            pl.BlockSpec((1, _DIM, _N), lambda b: (b, 0, 0)),
            pl.BlockSpec((_NPROJ, _DIM), lambda b: (0, 0)),
            pl.BlockSpec((_NPROJ, _DIM), lambda b: (0, 0)),
        ],
        out_specs=pl.BlockSpec((1, 1, 1), lambda b: (b, 0, 0)),
        out_shape=jax.ShapeDtypeStruct((_BATCH, 1, 1), jnp.float32),
        compiler_params=pltpu.CompilerParams(
            dimension_semantics=("parallel",)),
    )(pt, qt, pa, pb)
    return jnp.sum(losses)


# speedup vs baseline: 17.8529x; 1.1950x over previous
import numpy as np

import jax
import jax.numpy as jnp
from jax.experimental import pallas as pl
from jax.experimental.pallas import tpu as pltpu

_NPROJ = 50
_N = 256
_DIM = 3
_BATCH = 8


def _projections():
    z = jax.random.normal(jax.random.key(42), (_NPROJ, _DIM, 2), dtype=jnp.float32)
    q, _ = jnp.linalg.qr(z)
    return q


try:
    _PROJ_CONST = np.asarray(_projections())
except Exception:
    _PROJ_CONST = None


def _bitonic_sort_cols(x):
    n = x.shape[0]
    row = jax.lax.broadcasted_iota(jnp.int32, x.shape, 0)
    k = 2
    while k <= n:
        j = k // 2
        while j >= 1:
            upper = (row & j) != 0
            partner = jnp.where(upper, jnp.roll(x, j, axis=0),
                                jnp.roll(x, -j, axis=0))
            asc = (row & k) == 0
            take_min = asc != upper
            x = jnp.where(take_min, jnp.minimum(x, partner),
                          jnp.maximum(x, partner))
            j //= 2
        k *= 2
    return x


def _ssw_kernel(pt_ref, qt_ref, pab_ref, out_ref):
    pab = pab_ref[...]

    def coords(p):
        x = jax.lax.dot(p, pab, preferred_element_type=jnp.float32)
        return (jnp.pi + jnp.arctan2(-x[:, _NPROJ:], -x[:, :_NPROJ])) \
            * (1.0 / (2.0 * jnp.pi))

    ones = jnp.ones((_N, 1), jnp.float32)

    def shift_costs(ur, vb, shift, stride):
        circ = pltpu.roll(vb, shift, axis=1, stride=stride, stride_axis=0)
        diff = jnp.abs(ur - circ)
        d = jnp.minimum(diff, 1.0 - diff)
        return jax.lax.dot(d * d, ones,
                           preferred_element_type=jnp.float32)

    uvt = _bitonic_sort_cols(
        jnp.concatenate([coords(pt_ref[0]), coords(qt_ref[0])], axis=1))
    uv = uvt.T
    u = uv[:_NPROJ]
    v = uv[_NPROJ:]

    total = jnp.zeros((1, 1), jnp.float32)
    for l in range(_NPROJ):
        vb = jnp.broadcast_to(v[l:l + 1, :], (_N, _N))
        fr = shift_costs(u[l:l + 1, :], vb, 0, 1)
        total = total + jnp.min(fr, axis=(0, 1), keepdims=True)
    out_ref[0] = jnp.sqrt(total * (1.0 / (_NPROJ * _N)))


def kernel(P_batch, Q_batch):
    proj = _projections() if _PROJ_CONST is None else jnp.asarray(_PROJ_CONST)
    pa = proj[:, :, 0]
    pb = proj[:, :, 1]
    pab = jnp.concatenate([pa, pb], axis=0).T

    losses = pl.pallas_call(
        _ssw_kernel,
        grid=(_BATCH,),
        in_specs=[
            pl.BlockSpec((1, _N, _DIM), lambda b: (b, 0, 0)),
            pl.BlockSpec((1, _N, _DIM), lambda b: (b, 0, 0)),
            pl.BlockSpec((_DIM, 2 * _NPROJ), lambda b: (0, 0)),
        ],
        out_specs=pl.BlockSpec((1, 1, 1), lambda b: (b, 0, 0)),
        out_shape=jax.ShapeDtypeStruct((_BATCH, 1, 1), jnp.float32),
        compiler_params=pltpu.CompilerParams(
            dimension_semantics=("parallel",)),
    )(P_batch, Q_batch, pab)
    return jnp.sum(losses)
